# baseline (speedup 1.0000x reference)
import functools

import jax
import jax.numpy as jnp
from jax import lax
from jax.experimental import pallas as pl
from jax.experimental.pallas import tpu as pltpu
from jax.experimental.pallas import tpu_sc as plsc

NUM_T = 1000
GAMMA_PAD = 1001
B = 16384
L = 16

_info = plsc.get_sparse_core_info()
_NC, _NS = 1, _info.num_subcores
NW = _NC * _NS
B_PER_W = B // NW
VECS = B_PER_W // L


@functools.partial(
    pl.kernel,
    mesh=plsc.VectorSubcoreMesh(
        core_axis_name="c", subcore_axis_name="s", num_cores=_NC
    ),
    out_type=jax.ShapeDtypeStruct((B,), jnp.float32),
    scratch_types=[
        pltpu.VMEM((B_PER_W,), jnp.float32),
        pltpu.VMEM((GAMMA_PAD,), jnp.float32),
        pltpu.SemaphoreType.DMA,
    ],
    compiler_params=pltpu.CompilerParams(needs_layout_passes=False),
)
def _sc_lookup(t_hbm, gamma_hbm, out_hbm, t_v, gamma_v, sem):
    base = lax.axis_index("s") * B_PER_W
    cp_g = pltpu.async_copy(gamma_hbm, gamma_v, sem)
    cp_t = pltpu.async_copy(t_hbm.at[pl.ds(base, B_PER_W)], t_v, sem)
    cp_g.wait()
    cp_t.wait()
    magic = jnp.float32(12582912.0)

    def body(i, _):
        off = i * L
        tv = t_v[pl.ds(off, L)]
        y = tv * jnp.float32(NUM_T)
        idx = ((y + magic) - magic).astype(jnp.int32)
        t_v[pl.ds(off, L)] = plsc.load_gather(gamma_v, [idx])
        return 0

    lax.fori_loop(0, VECS, body, 0)
    pltpu.sync_copy(t_v, out_hbm.at[pl.ds(base, B_PER_W)])


def kernel(t, gamma):
    return _sc_lookup(t.reshape(B), gamma).reshape(B, 1)


# speedup vs baseline: 1.0373x; 1.0373x over previous
import functools

import jax
import jax.numpy as jnp
from jax import lax
from jax.experimental import pallas as pl
from jax.experimental.pallas import tpu as pltpu
from jax.experimental.pallas import tpu_sc as plsc

NUM_T = 1000
GAMMA_PAD = 1001
B = 16384
L = 16

_info = plsc.get_sparse_core_info()
_NC, _NS = 1, _info.num_subcores
NW = _NC * _NS
B_PER_W = B // NW
VECS = B_PER_W // L


@functools.partial(
    pl.kernel,
    mesh=plsc.VectorSubcoreMesh(
        core_axis_name="c", subcore_axis_name="s", num_cores=_NC
    ),
    out_type=jax.ShapeDtypeStruct((B,), jnp.float32),
    scratch_types=[
        pltpu.VMEM((B_PER_W,), jnp.float32),
        pltpu.VMEM((GAMMA_PAD,), jnp.float32),
        pltpu.VMEM((B_PER_W,), jnp.float32),
        pltpu.SemaphoreType.DMA,
    ],
    compiler_params=pltpu.CompilerParams(needs_layout_passes=False),
)
def _sc_lookup(t_hbm, gamma_hbm, out_hbm, t_v, gamma_v, out_v, sem):
    base = lax.axis_index("s") * B_PER_W
    cp_g = pltpu.async_copy(gamma_hbm, gamma_v, sem)
    cp_t = pltpu.async_copy(t_hbm.at[pl.ds(base, B_PER_W)], t_v, sem)
    cp_g.wait()
    cp_t.wait()
    magic = jnp.float32(12582912.0)

    def body(i, _):
        off = i * L
        tv = t_v[pl.ds(off, L)]
        y = tv * jnp.float32(NUM_T)
        idx = ((y + magic) - magic).astype(jnp.int32)
        out_v[pl.ds(off, L)] = plsc.load_gather(gamma_v, [idx])
        return 0

    lax.fori_loop(0, VECS, body, 0)
    pltpu.sync_copy(out_v, out_hbm.at[pl.ds(base, B_PER_W)])


def kernel(t, gamma):
    return _sc_lookup(t.reshape(B), gamma).reshape(B, 1)
